# baseline (speedup 1.0000x reference)
import functools
import numpy as np
import jax
import jax.numpy as jnp
from jax import lax
from jax.experimental import pallas as pl
from jax.experimental.pallas import tpu as pltpu
from jax.experimental.pallas import tpu_sc as plsc

EMBED = 256
G = 8
L = 4
NC = 6
NLEARN = 6
FIX_SCALE = np.array([[0.0, 0.0, 0.0], [0.45, 0.0, 0.0], [-0.45, 0.0, 0.0],
                      [0.0, 0.45, 0.0], [0.0, -0.45, 0.0], [0.0, 0.0, 0.45],
                      [0.0, 0.0, -0.45]], dtype=np.float32)
NPTS = FIX_SCALE.shape[0] + NLEARN
FM_SHAPES = ((64, 176), (32, 88), (16, 44), (8, 22))
HW_SIZES = tuple(h * w for h, w in FM_SHAPES)
LEVEL_OFF = (0, 11264, 14080, 14784)
ROWS_PER_CAM = 14960
TOT_ROWS = NC * ROWS_PER_CAM

NWORK = 32
APW = 30
NA_PAD = NWORK * APW
S_TOT = NC * L * NPTS
CH_SAMP = 26
NCHUNK = S_TOT // CH_SAMP
CH_ROWS = CH_SAMP * 4
NSPLIT = 13
SPART = CH_ROWS // NSPLIT

PERM = np.arange(256).reshape(16, 16).T.reshape(-1)
_PAIR = np.repeat(np.arange(8), 2).astype(np.int32)


def _sc_fuse_body(table, idxh, bilh, attnh, out,
                  idx_v0, idx_v1, bil_v0, bil_v1, attn_v0, attn_v1,
                  gb0, gb1, gb2, out_v,
                  sem_m0, sem_m1, sem_g0, sem_g1, sem_g2):
    cid = lax.axis_index("c")
    sid = lax.axis_index("s")
    wid = sid * 2 + cid
    base_n = wid * APW

    pair_const = lax.iota(jnp.int32, 16) >> 1
    zero16 = jnp.zeros((16,), jnp.float32)
    gbufs = (gb0, gb1, gb2)
    gsems = (sem_g0, sem_g1, sem_g2)

    def fire_meta(a, idx_v, bil_v, attn_v, sem):
        n = base_n + a
        pltpu.async_copy(idxh.at[n], idx_v, sem)
        pltpu.async_copy(bilh.at[n], bil_v, sem)
        pltpu.async_copy(attnh.at[n], attn_v, sem)

    def drain_meta(idx_v, bil_v, attn_v, sem):
        pltpu.make_async_copy(idxh.at[0], idx_v, sem).wait()
        pltpu.make_async_copy(bilh.at[0], bil_v, sem).wait()
        pltpu.make_async_copy(attnh.at[0], attn_v, sem).wait()

    def fire_gather(idx_v, ch, k):
        for m in range(NSPLIT):
            pltpu.async_copy(
                table.at[idx_v.at[ch * NSPLIT + m]],
                gbufs[k].at[pl.ds(m * SPART, SPART)], gsems[k])

    def drain_gather(k):
        for m in range(NSPLIT):
            pltpu.make_async_copy(
                table.at[pl.ds(0, SPART)],
                gbufs[k].at[pl.ds(m * SPART, SPART)], gsems[k]).wait()

    def chunk_compute(buf, bil_v, attn_v, ch, acc):
        base_s = ch * CH_SAMP

        def sbody(s, acc):
            smp = base_s + s
            smp16 = jnp.full((16,), smp, jnp.int32)
            wv = plsc.load_gather(attn_v, [smp16, pair_const])
            b0 = plsc.load_gather(bil_v, [smp16, jnp.full((16,), 0, jnp.int32)])
            b1 = plsc.load_gather(bil_v, [smp16, jnp.full((16,), 1, jnp.int32)])
            b2 = plsc.load_gather(bil_v, [smp16, jnp.full((16,), 2, jnp.int32)])
            b3 = plsc.load_gather(bil_v, [smp16, jnp.full((16,), 3, jnp.int32)])
            r = s * 4
            new = []
            for j in range(16):
                sl = pl.ds(16 * j, 16)
                p = (b0 * buf[r, sl] + b1 * buf[r + 1, sl]
                     + b2 * buf[r + 2, sl] + b3 * buf[r + 3, sl])
                new.append(acc[j] + wv * p)
            return tuple(new)

        return lax.fori_loop(0, CH_SAMP, sbody, acc)

    def do_anchor(a, idx_v, bil_v, attn_v, sem_m):
        n = base_n + a
        drain_meta(idx_v, bil_v, attn_v, sem_m)
        for k in range(3):
            fire_gather(idx_v, k, k)

        def rbody(r, acc):
            for k in range(3):
                ch = 3 * r + k
                drain_gather(k)
                acc = chunk_compute(gbufs[k], bil_v, attn_v, ch, acc)

                @pl.when(r < 3)
                def _():
                    fire_gather(idx_v, ch + 3, k)
            return acc

        acc = lax.fori_loop(0, NCHUNK // 3, rbody, (zero16,) * 16)
        for j in range(16):
            out_v[pl.ds(16 * j, 16)] = acc[j]
        pltpu.sync_copy(out_v, out.at[n])

    fire_meta(0, idx_v0, bil_v0, attn_v0, sem_m0)
    fire_meta(1, idx_v1, bil_v1, attn_v1, sem_m1)

    def pbody(p, _):
        a0 = 2 * p
        do_anchor(a0, idx_v0, bil_v0, attn_v0, sem_m0)

        @pl.when(p < APW // 2 - 1)
        def _():
            fire_meta(a0 + 2, idx_v0, bil_v0, attn_v0, sem_m0)

        do_anchor(a0 + 1, idx_v1, bil_v1, attn_v1, sem_m1)

        @pl.when(p < APW // 2 - 1)
        def _():
            fire_meta(a0 + 3, idx_v1, bil_v1, attn_v1, sem_m1)

        return 0

    lax.fori_loop(0, APW // 2, pbody, 0)


@jax.jit
def _sc_fuse(table, idxh, bilh, attnh):
    return pl.kernel(
        _sc_fuse_body,
        out_type=jax.ShapeDtypeStruct((NA_PAD, EMBED), jnp.float32),
        mesh=plsc.VectorSubcoreMesh(core_axis_name="c", subcore_axis_name="s"),
        compiler_params=pltpu.CompilerParams(
            use_tc_tiling_on_sc=False, needs_layout_passes=False),
        scratch_types=[
            pltpu.VMEM((NCHUNK * NSPLIT, SPART), jnp.int32),
            pltpu.VMEM((NCHUNK * NSPLIT, SPART), jnp.int32),
            pltpu.VMEM((S_TOT, 4), jnp.float32),
            pltpu.VMEM((S_TOT, 4), jnp.float32),
            pltpu.VMEM((S_TOT, G), jnp.float32),
            pltpu.VMEM((S_TOT, G), jnp.float32),
            pltpu.VMEM((CH_ROWS, EMBED), jnp.float32),
            pltpu.VMEM((CH_ROWS, EMBED), jnp.float32),
            pltpu.VMEM((CH_ROWS, EMBED), jnp.float32),
            pltpu.VMEM((EMBED,), jnp.float32),
            pltpu.SemaphoreType.DMA,
            pltpu.SemaphoreType.DMA,
            pltpu.SemaphoreType.DMA,
            pltpu.SemaphoreType.DMA,
            pltpu.SemaphoreType.DMA,
        ],
    )(table, idxh, bilh, attnh)


def _epilogue_body(f_ref, w_ref, b_ref, res_ref, o_ref):
    o_ref[...] = (jnp.dot(f_ref[...], w_ref[...], preferred_element_type=jnp.float32)
                  + b_ref[...] + res_ref[...])


def _epilogue(f, w, b, res):
    na, e = f.shape
    return pl.pallas_call(
        _epilogue_body,
        out_shape=jax.ShapeDtypeStruct((na, e), jnp.float32),
    )(f, w, b[None, :], res)


def kernel(instance_feature, anchor, anchor_embed, feature_map_0, feature_map_1,
           feature_map_2, feature_map_3, projection_mat, image_wh,
           learnable_fc_w, learnable_fc_b, weights_fc_w, weights_fc_b,
           output_proj_w, output_proj_b):
    bs, na = instance_feature.shape[:2]

    parts = []
    for fm, hw in zip((feature_map_0, feature_map_1, feature_map_2, feature_map_3),
                      HW_SIZES):
        parts.append(fm.reshape(NC, EMBED, hw))
    t = jnp.concatenate(parts, axis=2)
    t = t.reshape(NC, 16, 16, ROWS_PER_CAM).transpose(0, 3, 2, 1)
    table = t.reshape(TOT_ROWS, EMBED)

    fix = jnp.asarray(FIX_SCALE)
    scale = jnp.broadcast_to(fix[None, None], (bs, na, fix.shape[0], 3))
    learn = jax.nn.sigmoid(instance_feature @ learnable_fc_w + learnable_fc_b).reshape(bs, na, NLEARN, 3) - 0.5
    scale = jnp.concatenate([scale, learn], axis=2)
    kp = scale * jnp.exp(anchor[:, :, None, 3:6])
    sn = anchor[..., 6]
    cs = anchor[..., 7]
    zz = jnp.zeros_like(sn)
    oo = jnp.ones_like(sn)
    R = jnp.stack([cs, -sn, zz, sn, cs, zz, zz, zz, oo], axis=-1).reshape(bs, na, 3, 3)
    kp = jnp.einsum('bnij,bnpj->bnpi', R, kp) + anchor[:, :, None, :3]

    feat = instance_feature + anchor_embed
    w = (feat @ weights_fc_w + weights_fc_b).reshape(bs, na, -1, G)
    w = jax.nn.softmax(w, axis=-2)
    attn = w.reshape(na, S_TOT, G)

    pts4 = jnp.concatenate([kp, jnp.ones_like(kp[..., :1])], axis=-1)
    p2d = jnp.einsum('bcij,bnpj->bcnpi', projection_mat, pts4)
    p2d = p2d[..., :2] / jnp.maximum(p2d[..., 2:3], 1e-5)
    p2d = p2d / image_wh[:, :, None, None, :]
    px = p2d[0, ..., 0]
    py = p2d[0, ..., 1]

    cam_base = (jnp.arange(NC, dtype=jnp.int32) * ROWS_PER_CAM)[:, None, None]
    idx_l, bil_l = [], []
    for l, (H, W) in enumerate(FM_SHAPES):
        x = px * W - 0.5
        y = py * H - 0.5
        x0 = jnp.floor(x)
        y0 = jnp.floor(y)
        wx1 = x - x0
        wx0 = 1.0 - wx1
        wy1 = y - y0
        wy0 = 1.0 - wy1
        rows_t, wts_t = [], []
        for dx, dy in ((0, 0), (1, 0), (0, 1), (1, 1)):
            xf = x0 + dx
            yf = y0 + dy
            wt = (wx1 if dx else wx0) * (wy1 if dy else wy0)
            valid = (xf >= 0) & (xf <= W - 1) & (yf >= 0) & (yf <= H - 1)
            ixi = jnp.clip(xf, 0, W - 1).astype(jnp.int32)
            iyi = jnp.clip(yf, 0, H - 1).astype(jnp.int32)
            row = cam_base + LEVEL_OFF[l] + iyi * W + ixi
            rows_t.append(jnp.where(valid, row, 0))
            wts_t.append(jnp.where(valid, wt, 0.0))
        idx_l.append(jnp.stack(rows_t, axis=-1))
        bil_l.append(jnp.stack(wts_t, axis=-1))
    idx = jnp.stack(idx_l, axis=1)
    bil = jnp.stack(bil_l, axis=1)
    idx = idx.transpose(2, 0, 1, 3, 4).reshape(na, S_TOT * 4)
    bil = bil.transpose(2, 0, 1, 3, 4).reshape(na, S_TOT, 4)

    pad = NA_PAD - na
    idxh = jnp.pad(idx, ((0, pad), (0, 0))).reshape(NA_PAD, NCHUNK * NSPLIT, SPART)
    bilh = jnp.pad(bil, ((0, pad), (0, 0), (0, 0)))
    attnh = jnp.pad(attn, ((0, pad), (0, 0), (0, 0)))

    f_perm = _sc_fuse(table, idxh, bilh, attnh)

    w_perm = output_proj_w[jnp.asarray(PERM), :]
    res = jnp.pad(instance_feature.reshape(na, EMBED), ((0, pad), (0, 0)))
    out = _epilogue(f_perm, w_perm, output_proj_b, res)
    return out[:na].reshape(bs, na, EMBED)


# speedup vs baseline: 1.3751x; 1.3751x over previous
import functools
import numpy as np
import jax
import jax.numpy as jnp
from jax import lax
from jax.experimental import pallas as pl
from jax.experimental.pallas import tpu as pltpu
from jax.experimental.pallas import tpu_sc as plsc

EMBED = 256
G = 8
L = 4
NC = 6
NLEARN = 6
FIX_SCALE = np.array([[0.0, 0.0, 0.0], [0.45, 0.0, 0.0], [-0.45, 0.0, 0.0],
                      [0.0, 0.45, 0.0], [0.0, -0.45, 0.0], [0.0, 0.0, 0.45],
                      [0.0, 0.0, -0.45]], dtype=np.float32)
NPTS = FIX_SCALE.shape[0] + NLEARN
FM_SHAPES = ((64, 176), (32, 88), (16, 44), (8, 22))
HW_SIZES = tuple(h * w for h, w in FM_SHAPES)
LEVEL_OFF = (0, 11264, 14080, 14784)
ROWS_PER_CAM = 14960
TOT_ROWS = NC * ROWS_PER_CAM

NWORK = 32
APW = 30
NA_PAD = NWORK * APW
S_TOT = NC * L * NPTS
CH_SAMP = 26
NCHUNK = S_TOT // CH_SAMP
CH_ROWS = CH_SAMP * 2
WIDE = 2 * EMBED
TOT2 = TOT_ROWS + 1

PERM = np.arange(256).reshape(16, 16).T.reshape(-1)
_PAIR = np.repeat(np.arange(8), 2).astype(np.int32)


def _sc_fuse_body(table, idxh, bilh, attnh, out,
                  idx_v0, idx_v1, bil_v0, bil_v1, attn_v0, attn_v1,
                  gb0, gb1, gb2, out_v,
                  sem_m0, sem_m1, sem_g0, sem_g1, sem_g2):
    cid = lax.axis_index("c")
    sid = lax.axis_index("s")
    wid = sid * 2 + cid
    base_n = wid * APW

    pair_const = lax.iota(jnp.int32, 16) >> 1
    zero16 = jnp.zeros((16,), jnp.float32)
    gbufs = (gb0, gb1, gb2)
    gsems = (sem_g0, sem_g1, sem_g2)

    def fire_meta(a, idx_v, bil_v, attn_v, sem):
        n = base_n + a
        pltpu.async_copy(idxh.at[n], idx_v, sem)
        pltpu.async_copy(bilh.at[n], bil_v, sem)
        pltpu.async_copy(attnh.at[n], attn_v, sem)

    def drain_meta(idx_v, bil_v, attn_v, sem):
        pltpu.make_async_copy(idxh.at[0], idx_v, sem).wait()
        pltpu.make_async_copy(bilh.at[0], bil_v, sem).wait()
        pltpu.make_async_copy(attnh.at[0], attn_v, sem).wait()

    def fire_gather(idx_v, ch, k):
        pltpu.async_copy(table.at[idx_v.at[ch]], gbufs[k], gsems[k])

    def drain_gather(k):
        pltpu.make_async_copy(table.at[pl.ds(0, CH_ROWS)], gbufs[k], gsems[k]).wait()

    def chunk_compute(buf, bil_v, attn_v, ch, acc):
        base_s = ch * CH_SAMP

        def sbody(s, acc):
            smp = base_s + s
            smp16 = jnp.full((16,), smp, jnp.int32)
            wv = plsc.load_gather(attn_v, [smp16, pair_const])
            b0 = plsc.load_gather(bil_v, [smp16, jnp.full((16,), 0, jnp.int32)])
            b1 = plsc.load_gather(bil_v, [smp16, jnp.full((16,), 1, jnp.int32)])
            b2 = plsc.load_gather(bil_v, [smp16, jnp.full((16,), 2, jnp.int32)])
            b3 = plsc.load_gather(bil_v, [smp16, jnp.full((16,), 3, jnp.int32)])
            r = s * 2
            new = []
            for j in range(16):
                sl0 = pl.ds(16 * j, 16)
                sl1 = pl.ds(EMBED + 16 * j, 16)
                p = (b0 * buf[r, sl0] + b1 * buf[r, sl1]
                     + b2 * buf[r + 1, sl0] + b3 * buf[r + 1, sl1])
                new.append(acc[j] + wv * p)
            return tuple(new)

        return lax.fori_loop(0, CH_SAMP, sbody, acc)

    def do_anchor(a, idx_v, bil_v, attn_v, sem_m):
        n = base_n + a
        drain_meta(idx_v, bil_v, attn_v, sem_m)
        for k in range(3):
            fire_gather(idx_v, k, k)

        def rbody(r, acc):
            for k in range(3):
                ch = 3 * r + k
                drain_gather(k)
                acc = chunk_compute(gbufs[k], bil_v, attn_v, ch, acc)

                @pl.when(r < 3)
                def _():
                    fire_gather(idx_v, ch + 3, k)
            return acc

        acc = lax.fori_loop(0, NCHUNK // 3, rbody, (zero16,) * 16)
        for j in range(16):
            out_v[pl.ds(16 * j, 16)] = acc[j]
        pltpu.sync_copy(out_v, out.at[n])

    fire_meta(0, idx_v0, bil_v0, attn_v0, sem_m0)
    fire_meta(1, idx_v1, bil_v1, attn_v1, sem_m1)

    def pbody(p, _):
        a0 = 2 * p
        do_anchor(a0, idx_v0, bil_v0, attn_v0, sem_m0)

        @pl.when(p < APW // 2 - 1)
        def _():
            fire_meta(a0 + 2, idx_v0, bil_v0, attn_v0, sem_m0)

        do_anchor(a0 + 1, idx_v1, bil_v1, attn_v1, sem_m1)

        @pl.when(p < APW // 2 - 1)
        def _():
            fire_meta(a0 + 3, idx_v1, bil_v1, attn_v1, sem_m1)

        return 0

    lax.fori_loop(0, APW // 2, pbody, 0)


@jax.jit
def _sc_fuse(table, idxh, bilh, attnh):
    return pl.kernel(
        _sc_fuse_body,
        out_type=jax.ShapeDtypeStruct((NA_PAD, EMBED), jnp.float32),
        mesh=plsc.VectorSubcoreMesh(core_axis_name="c", subcore_axis_name="s"),
        compiler_params=pltpu.CompilerParams(
            use_tc_tiling_on_sc=False, needs_layout_passes=False),
        scratch_types=[
            pltpu.VMEM((NCHUNK, CH_ROWS), jnp.int32),
            pltpu.VMEM((NCHUNK, CH_ROWS), jnp.int32),
            pltpu.VMEM((S_TOT, 4), jnp.float32),
            pltpu.VMEM((S_TOT, 4), jnp.float32),
            pltpu.VMEM((S_TOT, G), jnp.float32),
            pltpu.VMEM((S_TOT, G), jnp.float32),
            pltpu.VMEM((CH_ROWS, WIDE), jnp.float32),
            pltpu.VMEM((CH_ROWS, WIDE), jnp.float32),
            pltpu.VMEM((CH_ROWS, WIDE), jnp.float32),
            pltpu.VMEM((EMBED,), jnp.float32),
            pltpu.SemaphoreType.DMA,
            pltpu.SemaphoreType.DMA,
            pltpu.SemaphoreType.DMA,
            pltpu.SemaphoreType.DMA,
            pltpu.SemaphoreType.DMA,
        ],
    )(table, idxh, bilh, attnh)


def _epilogue_body(f_ref, w_ref, b_ref, res_ref, o_ref):
    o_ref[...] = (jnp.dot(f_ref[...], w_ref[...], preferred_element_type=jnp.float32)
                  + b_ref[...] + res_ref[...])


def _epilogue(f, w, b, res):
    na, e = f.shape
    return pl.pallas_call(
        _epilogue_body,
        out_shape=jax.ShapeDtypeStruct((na, e), jnp.float32),
    )(f, w, b[None, :], res)


def kernel(instance_feature, anchor, anchor_embed, feature_map_0, feature_map_1,
           feature_map_2, feature_map_3, projection_mat, image_wh,
           learnable_fc_w, learnable_fc_b, weights_fc_w, weights_fc_b,
           output_proj_w, output_proj_b):
    bs, na = instance_feature.shape[:2]

    parts = []
    for fm, hw in zip((feature_map_0, feature_map_1, feature_map_2, feature_map_3),
                      HW_SIZES):
        parts.append(fm.reshape(NC, EMBED, hw))
    t = jnp.concatenate(parts, axis=2)
    t = t.reshape(NC, 16, 16, ROWS_PER_CAM).transpose(0, 3, 2, 1)
    table = t.reshape(TOT_ROWS, EMBED)
    zpad = jnp.zeros((1, EMBED), jnp.float32)
    tz = jnp.concatenate([zpad, table, zpad], axis=0)
    table2 = jnp.concatenate([tz[:-1], tz[1:]], axis=1)

    fix = jnp.asarray(FIX_SCALE)
    scale = jnp.broadcast_to(fix[None, None], (bs, na, fix.shape[0], 3))
    learn = jax.nn.sigmoid(instance_feature @ learnable_fc_w + learnable_fc_b).reshape(bs, na, NLEARN, 3) - 0.5
    scale = jnp.concatenate([scale, learn], axis=2)
    kp = scale * jnp.exp(anchor[:, :, None, 3:6])
    sn = anchor[..., 6]
    cs = anchor[..., 7]
    zz = jnp.zeros_like(sn)
    oo = jnp.ones_like(sn)
    R = jnp.stack([cs, -sn, zz, sn, cs, zz, zz, zz, oo], axis=-1).reshape(bs, na, 3, 3)
    kp = jnp.einsum('bnij,bnpj->bnpi', R, kp) + anchor[:, :, None, :3]

    feat = instance_feature + anchor_embed
    w = (feat @ weights_fc_w + weights_fc_b).reshape(bs, na, -1, G)
    w = jax.nn.softmax(w, axis=-2)
    attn = w.reshape(na, S_TOT, G)

    pts4 = jnp.concatenate([kp, jnp.ones_like(kp[..., :1])], axis=-1)
    p2d = jnp.einsum('bcij,bnpj->bcnpi', projection_mat, pts4)
    p2d = p2d[..., :2] / jnp.maximum(p2d[..., 2:3], 1e-5)
    p2d = p2d / image_wh[:, :, None, None, :]
    px = p2d[0, ..., 0]
    py = p2d[0, ..., 1]

    cam_base = (jnp.arange(NC, dtype=jnp.int32) * ROWS_PER_CAM)[:, None, None]
    idx_l, bil_l = [], []
    for l, (H, W) in enumerate(FM_SHAPES):
        x = px * W - 0.5
        y = py * H - 0.5
        x0 = jnp.floor(x)
        y0 = jnp.floor(y)
        wx1 = x - x0
        wx0 = 1.0 - wx1
        wy1 = y - y0
        wy0 = 1.0 - wy1
        vx0 = (x0 >= 0) & (x0 <= W - 1)
        vx1 = (x0 + 1 >= 0) & (x0 + 1 <= W - 1)
        vy0 = (y0 >= 0) & (y0 <= H - 1)
        vy1 = (y0 + 1 >= 0) & (y0 + 1 <= H - 1)
        x0s = jnp.clip(x0, -1, W - 1).astype(jnp.int32)
        y0s = jnp.clip(y0, 0, H - 1).astype(jnp.int32)
        y1s = jnp.clip(y0 + 1, 0, H - 1).astype(jnp.int32)
        base = cam_base + LEVEL_OFF[l]
        r0 = base + y0s * W + x0s + 1
        r1 = base + y1s * W + x0s + 1
        w00 = wx0 * wy0 * vx0 * vy0
        w10 = wx1 * wy0 * vx1 * vy0
        w01 = wx0 * wy1 * vx0 * vy1
        w11 = wx1 * wy1 * vx1 * vy1
        idx_l.append(jnp.stack([r0, r1], axis=-1))
        bil_l.append(jnp.stack([w00, w10, w01, w11], axis=-1))
    idx = jnp.stack(idx_l, axis=1)
    bil = jnp.stack(bil_l, axis=1)
    idx = idx.transpose(2, 0, 1, 3, 4).reshape(na, S_TOT * 2)
    bil = bil.transpose(2, 0, 1, 3, 4).reshape(na, S_TOT, 4)

    pad = NA_PAD - na
    idxh = jnp.pad(idx, ((0, pad), (0, 0))).reshape(NA_PAD, NCHUNK, CH_ROWS)
    bilh = jnp.pad(bil, ((0, pad), (0, 0), (0, 0)))
    attnh = jnp.pad(attn, ((0, pad), (0, 0), (0, 0)))

    f_perm = _sc_fuse(table2, idxh, bilh, attnh)

    w_perm = output_proj_w[jnp.asarray(PERM), :]
    res = jnp.pad(instance_feature.reshape(na, EMBED), ((0, pad), (0, 0)))
    out = _epilogue(f_perm, w_perm, output_proj_b, res)
    return out[:na].reshape(bs, na, EMBED)


# speedup vs baseline: 1.5149x; 1.1016x over previous
import functools
import numpy as np
import jax
import jax.numpy as jnp
from jax import lax
from jax.experimental import pallas as pl
from jax.experimental.pallas import tpu as pltpu
from jax.experimental.pallas import tpu_sc as plsc

EMBED = 256
G = 8
L = 4
NC = 6
NLEARN = 6
FIX_SCALE = np.array([[0.0, 0.0, 0.0], [0.45, 0.0, 0.0], [-0.45, 0.0, 0.0],
                      [0.0, 0.45, 0.0], [0.0, -0.45, 0.0], [0.0, 0.0, 0.45],
                      [0.0, 0.0, -0.45]], dtype=np.float32)
NPTS = FIX_SCALE.shape[0] + NLEARN
FM_SHAPES = ((64, 176), (32, 88), (16, 44), (8, 22))
HW_SIZES = tuple(h * w for h, w in FM_SHAPES)
LEVEL_OFF = (0, 11264, 14080, 14784)
ROWS_PER_CAM = 14960
TOT_ROWS = NC * ROWS_PER_CAM

NWORK = 32
APW = 30
NA_PAD = NWORK * APW
S_TOT = NC * L * NPTS
CH_SAMP = 26
NCHUNK = S_TOT // CH_SAMP
CH_ROWS = CH_SAMP * 2
WIDE = 2 * EMBED
TOT2 = TOT_ROWS + 1

_m = np.arange(256)
_i = (_m % 32) >> 1
_par = _m & 1
_j = _m // 32
PERM = ((_i >> 1) * 32 + _j * 4 + (_i & 1) * 2 + _par).astype(np.int32)
_v = np.arange(16)[:, None]
_lane = np.arange(16)[None, :]
PERM_OUT = PERM[(32 * (_v // 2) + 2 * _lane + (_v % 2))].reshape(-1)


def _sc_fuse_body(table, idxh, bilh, attnh, out,
                  idx_v0, idx_v1, bil_v0, bil_v1, attn_v0, attn_v1,
                  gb0, gb1, gb2, out_v,
                  sem_m0, sem_m1, sem_g0, sem_g1, sem_g2):
    cid = lax.axis_index("c")
    sid = lax.axis_index("s")
    wid = sid * 2 + cid
    base_n = wid * APW

    pair_const = lax.iota(jnp.int32, 16) >> 1
    zero16 = jnp.zeros((16,), jnp.float32)
    gbufs = (gb0, gb1, gb2)
    gsems = (sem_g0, sem_g1, sem_g2)

    def fire_meta(a, idx_v, bil_v, attn_v, sem):
        n = base_n + a
        pltpu.async_copy(idxh.at[n], idx_v, sem)
        pltpu.async_copy(bilh.at[n], bil_v, sem)
        pltpu.async_copy(attnh.at[n], attn_v, sem)

    def drain_meta(idx_v, bil_v, attn_v, sem):
        pltpu.make_async_copy(idxh.at[0], idx_v, sem).wait()
        pltpu.make_async_copy(bilh.at[0], bil_v, sem).wait()
        pltpu.make_async_copy(attnh.at[0], attn_v, sem).wait()

    def fire_gather(idx_v, ch, k):
        pltpu.async_copy(table.at[idx_v.at[ch]], gbufs[k], gsems[k])

    def drain_gather(k):
        pltpu.make_async_copy(table.at[pl.ds(0, CH_ROWS)], gbufs[k], gsems[k]).wait()

    def chunk_compute(buf, bil_v, attn_v, ch, acc):
        base_s = ch * CH_SAMP
        ILV = plsc.PackFormat.INTERLEAVED

        def sbody(s, acc):
            smp = base_s + s
            smp16 = jnp.full((16,), smp, jnp.int32)
            wv = plsc.load_gather(attn_v, [smp16, pair_const])
            b0 = plsc.load_gather(bil_v, [smp16, jnp.full((16,), 0, jnp.int32)])
            b1 = plsc.load_gather(bil_v, [smp16, jnp.full((16,), 1, jnp.int32)])
            b2 = plsc.load_gather(bil_v, [smp16, jnp.full((16,), 2, jnp.int32)])
            b3 = plsc.load_gather(bil_v, [smp16, jnp.full((16,), 3, jnp.int32)])
            r = s * 2
            new = []
            for k in range(8):
                sl0 = pl.ds(32 * k, 32)
                sl1 = pl.ds(EMBED + 32 * k, 32)
                q0a, q0b = plsc.unpack(buf[r, sl0], format=ILV)
                q1a, q1b = plsc.unpack(buf[r, sl1], format=ILV)
                q2a, q2b = plsc.unpack(buf[r + 1, sl0], format=ILV)
                q3a, q3b = plsc.unpack(buf[r + 1, sl1], format=ILV)
                pa = b0 * q0a + b1 * q1a + b2 * q2a + b3 * q3a
                pb = b0 * q0b + b1 * q1b + b2 * q2b + b3 * q3b
                new.append(acc[2 * k] + wv * pa)
                new.append(acc[2 * k + 1] + wv * pb)
            return tuple(new)

        return lax.fori_loop(0, CH_SAMP, sbody, acc)

    def do_anchor(a, idx_v, bil_v, attn_v, sem_m):
        n = base_n + a
        drain_meta(idx_v, bil_v, attn_v, sem_m)
        for k in range(3):
            fire_gather(idx_v, k, k)

        def rbody(r, acc):
            for k in range(3):
                ch = 3 * r + k
                drain_gather(k)
                acc = chunk_compute(gbufs[k], bil_v, attn_v, ch, acc)

                @pl.when(r < 3)
                def _():
                    fire_gather(idx_v, ch + 3, k)
            return acc

        acc = lax.fori_loop(0, NCHUNK // 3, rbody, (zero16,) * 16)
        for j in range(16):
            out_v[pl.ds(16 * j, 16)] = acc[j]
        pltpu.sync_copy(out_v, out.at[n])

    fire_meta(0, idx_v0, bil_v0, attn_v0, sem_m0)
    fire_meta(1, idx_v1, bil_v1, attn_v1, sem_m1)

    def pbody(p, _):
        a0 = 2 * p
        do_anchor(a0, idx_v0, bil_v0, attn_v0, sem_m0)

        @pl.when(p < APW // 2 - 1)
        def _():
            fire_meta(a0 + 2, idx_v0, bil_v0, attn_v0, sem_m0)

        do_anchor(a0 + 1, idx_v1, bil_v1, attn_v1, sem_m1)

        @pl.when(p < APW // 2 - 1)
        def _():
            fire_meta(a0 + 3, idx_v1, bil_v1, attn_v1, sem_m1)

        return 0

    lax.fori_loop(0, APW // 2, pbody, 0)


@jax.jit
def _sc_fuse(table, idxh, bilh, attnh):
    return pl.kernel(
        _sc_fuse_body,
        out_type=jax.ShapeDtypeStruct((NA_PAD, EMBED), jnp.float32),
        mesh=plsc.VectorSubcoreMesh(core_axis_name="c", subcore_axis_name="s"),
        compiler_params=pltpu.CompilerParams(
            use_tc_tiling_on_sc=False, needs_layout_passes=False),
        scratch_types=[
            pltpu.VMEM((NCHUNK, CH_ROWS), jnp.int32),
            pltpu.VMEM((NCHUNK, CH_ROWS), jnp.int32),
            pltpu.VMEM((S_TOT, 4), jnp.float32),
            pltpu.VMEM((S_TOT, 4), jnp.float32),
            pltpu.VMEM((S_TOT, G), jnp.float32),
            pltpu.VMEM((S_TOT, G), jnp.float32),
            pltpu.VMEM((CH_ROWS, WIDE), jnp.bfloat16),
            pltpu.VMEM((CH_ROWS, WIDE), jnp.bfloat16),
            pltpu.VMEM((CH_ROWS, WIDE), jnp.bfloat16),
            pltpu.VMEM((EMBED,), jnp.float32),
            pltpu.SemaphoreType.DMA,
            pltpu.SemaphoreType.DMA,
            pltpu.SemaphoreType.DMA,
            pltpu.SemaphoreType.DMA,
            pltpu.SemaphoreType.DMA,
        ],
    )(table, idxh, bilh, attnh)


def _epilogue_body(f_ref, w_ref, b_ref, res_ref, o_ref):
    o_ref[...] = (jnp.dot(f_ref[...], w_ref[...], preferred_element_type=jnp.float32)
                  + b_ref[...] + res_ref[...])


def _epilogue(f, w, b, res):
    na, e = f.shape
    return pl.pallas_call(
        _epilogue_body,
        out_shape=jax.ShapeDtypeStruct((na, e), jnp.float32),
    )(f, w, b[None, :], res)


def kernel(instance_feature, anchor, anchor_embed, feature_map_0, feature_map_1,
           feature_map_2, feature_map_3, projection_mat, image_wh,
           learnable_fc_w, learnable_fc_b, weights_fc_w, weights_fc_b,
           output_proj_w, output_proj_b):
    bs, na = instance_feature.shape[:2]

    parts = []
    for fm, hw in zip((feature_map_0, feature_map_1, feature_map_2, feature_map_3),
                      HW_SIZES):
        parts.append(fm.reshape(NC, EMBED, hw))
    t = jnp.concatenate(parts, axis=2)
    t = t.astype(jnp.bfloat16)[:, jnp.asarray(PERM), :].transpose(0, 2, 1)
    table = t.reshape(TOT_ROWS, EMBED)
    zpad = jnp.zeros((1, EMBED), jnp.bfloat16)
    tz = jnp.concatenate([zpad, table, zpad], axis=0)
    table2 = jnp.concatenate([tz[:-1], tz[1:]], axis=1)

    fix = jnp.asarray(FIX_SCALE)
    scale = jnp.broadcast_to(fix[None, None], (bs, na, fix.shape[0], 3))
    learn = jax.nn.sigmoid(instance_feature @ learnable_fc_w + learnable_fc_b).reshape(bs, na, NLEARN, 3) - 0.5
    scale = jnp.concatenate([scale, learn], axis=2)
    kp = scale * jnp.exp(anchor[:, :, None, 3:6])
    sn = anchor[..., 6]
    cs = anchor[..., 7]
    zz = jnp.zeros_like(sn)
    oo = jnp.ones_like(sn)
    R = jnp.stack([cs, -sn, zz, sn, cs, zz, zz, zz, oo], axis=-1).reshape(bs, na, 3, 3)
    kp = jnp.einsum('bnij,bnpj->bnpi', R, kp) + anchor[:, :, None, :3]

    feat = instance_feature + anchor_embed
    w = (feat @ weights_fc_w + weights_fc_b).reshape(bs, na, -1, G)
    w = jax.nn.softmax(w, axis=-2)
    attn = w.reshape(na, S_TOT, G)

    pts4 = jnp.concatenate([kp, jnp.ones_like(kp[..., :1])], axis=-1)
    p2d = jnp.einsum('bcij,bnpj->bcnpi', projection_mat, pts4)
    p2d = p2d[..., :2] / jnp.maximum(p2d[..., 2:3], 1e-5)
    p2d = p2d / image_wh[:, :, None, None, :]
    px = p2d[0, ..., 0]
    py = p2d[0, ..., 1]

    cam_base = (jnp.arange(NC, dtype=jnp.int32) * ROWS_PER_CAM)[:, None, None]
    idx_l, bil_l = [], []
    for l, (H, W) in enumerate(FM_SHAPES):
        x = px * W - 0.5
        y = py * H - 0.5
        x0 = jnp.floor(x)
        y0 = jnp.floor(y)
        wx1 = x - x0
        wx0 = 1.0 - wx1
        wy1 = y - y0
        wy0 = 1.0 - wy1
        vx0 = (x0 >= 0) & (x0 <= W - 1)
        vx1 = (x0 + 1 >= 0) & (x0 + 1 <= W - 1)
        vy0 = (y0 >= 0) & (y0 <= H - 1)
        vy1 = (y0 + 1 >= 0) & (y0 + 1 <= H - 1)
        x0s = jnp.clip(x0, -1, W - 1).astype(jnp.int32)
        y0s = jnp.clip(y0, 0, H - 1).astype(jnp.int32)
        y1s = jnp.clip(y0 + 1, 0, H - 1).astype(jnp.int32)
        base = cam_base + LEVEL_OFF[l]
        r0 = base + y0s * W + x0s + 1
        r1 = base + y1s * W + x0s + 1
        w00 = wx0 * wy0 * vx0 * vy0
        w10 = wx1 * wy0 * vx1 * vy0
        w01 = wx0 * wy1 * vx0 * vy1
        w11 = wx1 * wy1 * vx1 * vy1
        idx_l.append(jnp.stack([r0, r1], axis=-1))
        bil_l.append(jnp.stack([w00, w10, w01, w11], axis=-1))
    idx = jnp.stack(idx_l, axis=1)
    bil = jnp.stack(bil_l, axis=1)
    idx = idx.transpose(2, 0, 1, 3, 4).reshape(na, S_TOT * 2)
    bil = bil.transpose(2, 0, 1, 3, 4).reshape(na, S_TOT, 4)

    pad = NA_PAD - na
    idxh = jnp.pad(idx, ((0, pad), (0, 0))).reshape(NA_PAD, NCHUNK, CH_ROWS)
    bilh = jnp.pad(bil, ((0, pad), (0, 0), (0, 0)))
    attnh = jnp.pad(attn, ((0, pad), (0, 0), (0, 0)))

    f_perm = _sc_fuse(table2, idxh, bilh, attnh)

    w_perm = output_proj_w[jnp.asarray(PERM_OUT), :]
    res = jnp.pad(instance_feature.reshape(na, EMBED), ((0, pad), (0, 0)))
    out = _epilogue(f_perm, w_perm, output_proj_b, res)
    return out[:na].reshape(bs, na, EMBED)


# speedup vs baseline: 1.7705x; 1.1688x over previous
import functools
import numpy as np
import jax
import jax.numpy as jnp
from jax import lax
from jax.experimental import pallas as pl
from jax.experimental.pallas import tpu as pltpu
from jax.experimental.pallas import tpu_sc as plsc

EMBED = 256
G = 8
L = 4
NC = 6
NLEARN = 6
FIX_SCALE = np.array([[0.0, 0.0, 0.0], [0.45, 0.0, 0.0], [-0.45, 0.0, 0.0],
                      [0.0, 0.45, 0.0], [0.0, -0.45, 0.0], [0.0, 0.0, 0.45],
                      [0.0, 0.0, -0.45]], dtype=np.float32)
NPTS = FIX_SCALE.shape[0] + NLEARN
FM_SHAPES = ((64, 176), (32, 88), (16, 44), (8, 22))
HW_SIZES = tuple(h * w for h, w in FM_SHAPES)
LEVEL_OFF = (0, 11264, 14080, 14784)
ROWS_PER_CAM = 14960
TOT_ROWS = NC * ROWS_PER_CAM

NWORK = 32
APW = 30
NA_PAD = NWORK * APW
S_TOT = NC * L * NPTS
S_PAD = 384
CH_SAMP = 26
NCHUNK = S_TOT // CH_SAMP
CH_ROWS = CH_SAMP * 2
WIDE = 2 * EMBED
TOT2 = TOT_ROWS + 1

BLK = 128
NA_TC = 1024

_m = np.arange(256)
_i = (_m % 32) >> 1
_par = _m & 1
_j = _m // 32
PERM = ((_i >> 1) * 32 + _j * 4 + (_i & 1) * 2 + _par).astype(np.int32)
_v = np.arange(16)[:, None]
_lane = np.arange(16)[None, :]
PERM_OUT = PERM[(32 * (_v // 2) + 2 * _lane + (_v % 2))].reshape(-1)


def _sc_fuse_body(table, idxh, bilh, attnh, out,
                  idx_v0, idx_v1, bil_v0, bil_v1, attn_v0, attn_v1,
                  gb0a, gb0b, gb1a, gb1b, gb2a, gb2b, out_v,
                  sem_m0, sem_m1, sem_g0, sem_g1, sem_g2):
    cid = lax.axis_index("c")
    sid = lax.axis_index("s")
    wid = sid * 2 + cid
    base_n = wid * APW

    pair_const = lax.iota(jnp.int32, 16) >> 1
    zero16 = jnp.zeros((16,), jnp.float32)
    gbufs = ((gb0a, gb0b), (gb1a, gb1b), (gb2a, gb2b))
    gsems = (sem_g0, sem_g1, sem_g2)

    def fire_meta(a, idx_v, bil_v, attn_v, sem):
        n = base_n + a
        pltpu.async_copy(idxh.at[n], idx_v, sem)
        pltpu.async_copy(bilh.at[n], bil_v, sem)
        pltpu.async_copy(attnh.at[n], attn_v, sem)

    def drain_meta(idx_v, bil_v, attn_v, sem):
        pltpu.make_async_copy(idxh.at[0], idx_v, sem).wait()
        pltpu.make_async_copy(bilh.at[0], bil_v, sem).wait()
        pltpu.make_async_copy(attnh.at[0], attn_v, sem).wait()

    def fire_gather(idx_v, ch, k):
        pltpu.async_copy(table.at[idx_v.at[ch]], gbufs[k][0], gsems[k])
        pltpu.async_copy(table.at[idx_v.at[NCHUNK + ch]], gbufs[k][1], gsems[k])

    def drain_gather(k):
        pltpu.make_async_copy(table.at[pl.ds(0, CH_SAMP)], gbufs[k][0], gsems[k]).wait()
        pltpu.make_async_copy(table.at[pl.ds(0, CH_SAMP)], gbufs[k][1], gsems[k]).wait()

    def chunk_compute(buf, bil_v, attn_v, ch, acc):
        base_s = ch * CH_SAMP
        ILV = plsc.PackFormat.INTERLEAVED

        def sbody(s, acc):
            smp = base_s + s
            smp16 = jnp.full((16,), smp, jnp.int32)
            wv = plsc.load_gather(attn_v, [pair_const, smp16])
            b0 = plsc.load_gather(bil_v, [jnp.full((16,), 0, jnp.int32), smp16])
            b1 = plsc.load_gather(bil_v, [jnp.full((16,), 1, jnp.int32), smp16])
            b2 = plsc.load_gather(bil_v, [jnp.full((16,), 2, jnp.int32), smp16])
            b3 = plsc.load_gather(bil_v, [jnp.full((16,), 3, jnp.int32), smp16])
            buf0, buf1 = buf
            new = []
            for k in range(8):
                sl0 = pl.ds(32 * k, 32)
                sl1 = pl.ds(EMBED + 32 * k, 32)
                q0a, q0b = plsc.unpack(buf0[s, sl0], format=ILV)
                q1a, q1b = plsc.unpack(buf0[s, sl1], format=ILV)
                q2a, q2b = plsc.unpack(buf1[s, sl0], format=ILV)
                q3a, q3b = plsc.unpack(buf1[s, sl1], format=ILV)
                pa = b0 * q0a + b1 * q1a + b2 * q2a + b3 * q3a
                pb = b0 * q0b + b1 * q1b + b2 * q2b + b3 * q3b
                new.append(acc[2 * k] + wv * pa)
                new.append(acc[2 * k + 1] + wv * pb)
            return tuple(new)

        return lax.fori_loop(0, CH_SAMP, sbody, acc)

    def do_anchor(a, idx_v, bil_v, attn_v, sem_m):
        n = base_n + a
        drain_meta(idx_v, bil_v, attn_v, sem_m)
        for k in range(3):
            fire_gather(idx_v, k, k)

        def rbody(r, acc):
            for k in range(3):
                ch = 3 * r + k
                drain_gather(k)
                acc = chunk_compute(gbufs[k], bil_v, attn_v, ch, acc)

                @pl.when(r < 3)
                def _():
                    fire_gather(idx_v, ch + 3, k)
            return acc

        acc = lax.fori_loop(0, NCHUNK // 3, rbody, (zero16,) * 16)
        for j in range(16):
            out_v[pl.ds(16 * j, 16)] = acc[j]
        pltpu.sync_copy(out_v, out.at[n])

    fire_meta(0, idx_v0, bil_v0, attn_v0, sem_m0)
    fire_meta(1, idx_v1, bil_v1, attn_v1, sem_m1)

    def pbody(p, _):
        a0 = 2 * p
        do_anchor(a0, idx_v0, bil_v0, attn_v0, sem_m0)

        @pl.when(p < APW // 2 - 1)
        def _():
            fire_meta(a0 + 2, idx_v0, bil_v0, attn_v0, sem_m0)

        do_anchor(a0 + 1, idx_v1, bil_v1, attn_v1, sem_m1)

        @pl.when(p < APW // 2 - 1)
        def _():
            fire_meta(a0 + 3, idx_v1, bil_v1, attn_v1, sem_m1)

        return 0

    lax.fori_loop(0, APW // 2, pbody, 0)


@jax.jit
def _sc_fuse(table, idxh, bilh, attnh):
    return pl.kernel(
        _sc_fuse_body,
        out_type=jax.ShapeDtypeStruct((NA_PAD, EMBED), jnp.float32),
        mesh=plsc.VectorSubcoreMesh(core_axis_name="c", subcore_axis_name="s"),
        compiler_params=pltpu.CompilerParams(
            use_tc_tiling_on_sc=False, needs_layout_passes=False),
        scratch_types=[
            pltpu.VMEM((2 * NCHUNK, CH_SAMP), jnp.int32),
            pltpu.VMEM((2 * NCHUNK, CH_SAMP), jnp.int32),
            pltpu.VMEM((4, S_TOT), jnp.float32),
            pltpu.VMEM((4, S_TOT), jnp.float32),
            pltpu.VMEM((G, S_PAD), jnp.float32),
            pltpu.VMEM((G, S_PAD), jnp.float32),
            pltpu.VMEM((CH_SAMP, WIDE), jnp.bfloat16),
            pltpu.VMEM((CH_SAMP, WIDE), jnp.bfloat16),
            pltpu.VMEM((CH_SAMP, WIDE), jnp.bfloat16),
            pltpu.VMEM((CH_SAMP, WIDE), jnp.bfloat16),
            pltpu.VMEM((CH_SAMP, WIDE), jnp.bfloat16),
            pltpu.VMEM((CH_SAMP, WIDE), jnp.bfloat16),
            pltpu.VMEM((EMBED,), jnp.float32),
            pltpu.SemaphoreType.DMA,
            pltpu.SemaphoreType.DMA,
            pltpu.SemaphoreType.DMA,
            pltpu.SemaphoreType.DMA,
            pltpu.SemaphoreType.DMA,
        ],
    )(table, idxh, bilh, attnh)


def _prologue_body(if_ref, ae_ref, anc_ref, wl_ref, bl_ref, ww_ref, bw_ref,
                   cf_ref, ci_ref, idx0_ref, idx1_ref, b00_ref, b10_ref,
                   b01_ref, b11_ref, attn_ref):
    insf = if_ref[...]
    anc = anc_ref[...]

    ll = (jnp.dot(insf, wl_ref[...], preferred_element_type=jnp.float32)
          + bl_ref[...])
    den = 1.0 + jnp.exp(-ll)
    rd = 1.0 / den
    rd = rd * (2.0 - den * rd)
    rd = rd * (2.0 - den * rd)
    learn = rd - 0.5

    es = jnp.exp(anc[:, 3:6])
    kx = learn[:, 0:13] * es[:, 0:1]
    ky = learn[:, 13:26] * es[:, 1:2]
    kz = learn[:, 26:39] * es[:, 2:3]
    sn = anc[:, 6:7]
    cs = anc[:, 7:8]
    kxr = cs * kx - sn * ky + anc[:, 0:1]
    kyr = sn * kx + cs * ky + anc[:, 1:2]
    kzr = kz + anc[:, 2:3]

    KX = jnp.concatenate([kxr] * (NC * L), axis=1)
    KY = jnp.concatenate([kyr] * (NC * L), axis=1)
    KZ = jnp.concatenate([kzr] * (NC * L), axis=1)

    cf = cf_ref[...]
    xn = cf[0:1] * KX + cf[1:2] * KY + cf[2:3] * KZ + cf[3:4]
    yn = cf[4:5] * KX + cf[5:6] * KY + cf[6:7] * KZ + cf[7:8]
    zn = cf[8:9] * KX + cf[9:10] * KY + cf[10:11] * KZ + cf[11:12]
    z = jnp.maximum(zn, 1e-5)
    r = 1.0 / z
    r = r * (2.0 - z * r)
    r = r * (2.0 - z * r)
    x = (xn * r) * cf[12:13] - 0.5
    y = (yn * r) * cf[13:14] - 0.5
    Wv = cf[14:15]
    Hv = cf[15:16]

    xi = x.astype(jnp.int32)
    yi = y.astype(jnp.int32)
    x0i = xi - (xi.astype(jnp.float32) > x).astype(jnp.int32)
    y0i = yi - (yi.astype(jnp.float32) > y).astype(jnp.int32)
    x0 = x0i.astype(jnp.float32)
    y0 = y0i.astype(jnp.float32)
    wx1 = x - x0
    wx0 = 1.0 - wx1
    wy1 = y - y0
    wy0 = 1.0 - wy1
    vx0 = ((x0 >= 0) & (x0 <= Wv - 1)).astype(jnp.float32)
    vx1 = ((x0 >= -1) & (x0 <= Wv - 2)).astype(jnp.float32)
    vy0 = ((y0 >= 0) & (y0 <= Hv - 1)).astype(jnp.float32)
    vy1 = ((y0 >= -1) & (y0 <= Hv - 2)).astype(jnp.float32)
    base1 = ci_ref[0:1, :]
    WI = ci_ref[1:2, :]
    HI = ci_ref[2:3, :]
    x0s = jnp.clip(x0i, -1, WI - 1)
    y0s = jnp.clip(y0i, 0, HI - 1)
    y1s = jnp.clip(y0i + 1, 0, HI - 1)
    idx0_ref[...] = base1 + y0s * WI + x0s
    idx1_ref[...] = base1 + y1s * WI + x0s
    b00_ref[...] = wx0 * wy0 * vx0 * vy0
    b10_ref[...] = wx1 * wy0 * vx1 * vy0
    b01_ref[...] = wx0 * wy1 * vx0 * vy1
    b11_ref[...] = wx1 * wy1 * vx1 * vy1

    w2 = (jnp.dot(insf + ae_ref[...], ww_ref[...],
                  preferred_element_type=jnp.float32) + bw_ref[...])
    for g in range(G):
        seg = w2[:, g * S_PAD:(g + 1) * S_PAD]
        mx = jnp.max(seg, axis=1, keepdims=True)
        e = jnp.exp(seg - mx)
        attn_ref[:, g, :] = e / jnp.sum(e, axis=1, keepdims=True)


@jax.jit
def _prologue(if_p, ae_p, anc_p, wl_p, bl_p, ww_p, bw_p, cf, ci):
    grid = NA_TC // BLK
    return pl.pallas_call(
        _prologue_body,
        grid=(grid,),
        in_specs=[
            pl.BlockSpec((BLK, EMBED), lambda i: (i, 0)),
            pl.BlockSpec((BLK, EMBED), lambda i: (i, 0)),
            pl.BlockSpec((BLK, 16), lambda i: (i, 0)),
            pl.BlockSpec((EMBED, 128), lambda i: (0, 0)),
            pl.BlockSpec((1, 128), lambda i: (0, 0)),
            pl.BlockSpec((EMBED, G * S_PAD), lambda i: (0, 0)),
            pl.BlockSpec((1, G * S_PAD), lambda i: (0, 0)),
            pl.BlockSpec((16, S_TOT), lambda i: (0, 0)),
            pl.BlockSpec((3, S_TOT), lambda i: (0, 0)),
        ],
        out_specs=[pl.BlockSpec((BLK, S_TOT), lambda i: (i, 0))] * 6
        + [pl.BlockSpec((BLK, G, S_PAD), lambda i: (i, 0, 0))],
        out_shape=[jax.ShapeDtypeStruct((NA_TC, S_TOT), jnp.int32)] * 2
        + [jax.ShapeDtypeStruct((NA_TC, S_TOT), jnp.float32)] * 4
        + [jax.ShapeDtypeStruct((NA_TC, G, S_PAD), jnp.float32)],
    )(if_p, ae_p, anc_p, wl_p, bl_p, ww_p, bw_p, cf, ci)


def _epilogue_body(f_ref, w_ref, b_ref, res_ref, o_ref):
    o_ref[...] = (jnp.dot(f_ref[...], w_ref[...], preferred_element_type=jnp.float32)
                  + b_ref[...] + res_ref[...])


def _epilogue(f, w, b, res):
    na, e = f.shape
    return pl.pallas_call(
        _epilogue_body,
        out_shape=jax.ShapeDtypeStruct((na, e), jnp.float32),
    )(f, w, b[None, :], res)


def kernel(instance_feature, anchor, anchor_embed, feature_map_0, feature_map_1,
           feature_map_2, feature_map_3, projection_mat, image_wh,
           learnable_fc_w, learnable_fc_b, weights_fc_w, weights_fc_b,
           output_proj_w, output_proj_b):
    bs, na = instance_feature.shape[:2]
    f32 = jnp.float32

    parts = []
    for fm, hw in zip((feature_map_0, feature_map_1, feature_map_2, feature_map_3),
                      HW_SIZES):
        parts.append(fm.reshape(NC, EMBED, hw))
    t = jnp.concatenate(parts, axis=2)
    t = t.astype(jnp.bfloat16)[:, jnp.asarray(PERM), :].transpose(0, 2, 1)
    table = t.reshape(TOT_ROWS, EMBED)
    zpad = jnp.zeros((1, EMBED), jnp.bfloat16)
    tz = jnp.concatenate([zpad, table, zpad], axis=0)
    table2 = jnp.concatenate([tz[:-1], tz[1:]], axis=1)

    pad_tc = NA_TC - na
    if_p = jnp.pad(instance_feature.reshape(na, EMBED), ((0, pad_tc), (0, 0)))
    ae_p = jnp.pad(anchor_embed.reshape(na, EMBED), ((0, pad_tc), (0, 0)))
    anc_p = jnp.pad(anchor.reshape(na, -1), ((0, pad_tc), (0, 16 - anchor.shape[-1])))

    sel = np.zeros((NLEARN * 3, 3 * NPTS), np.float32)
    fixb = np.zeros((3 * NPTS,), np.float32)
    for c in range(3):
        for p in range(7):
            v = float(FIX_SCALE[p, c]) + 0.5
            fixb[c * NPTS + p] = float(np.log(v / (1.0 - v)))
        for q in range(NLEARN):
            sel[q * 3 + c, c * NPTS + 7 + q] = 1.0
    wl_p = jnp.pad(learnable_fc_w @ jnp.asarray(sel), ((0, 0), (0, 128 - 3 * NPTS)))
    bl_p = jnp.pad(learnable_fc_b @ jnp.asarray(sel) + jnp.asarray(fixb),
                   (0, 128 - 3 * NPTS))[None, :]

    ww3 = weights_fc_w.reshape(EMBED, S_TOT, G).transpose(0, 2, 1)
    ww_p = jnp.pad(ww3, ((0, 0), (0, 0), (0, S_PAD - S_TOT))).reshape(EMBED, G * S_PAD)
    bw3 = weights_fc_b.reshape(S_TOT, G).T
    bw_p = jnp.pad(bw3, ((0, 0), (0, S_PAD - S_TOT)),
                   constant_values=-1e30).reshape(1, G * S_PAD)

    lane = np.arange(S_TOT)
    cidx = jnp.asarray(lane // (L * NPTS))
    lidx = jnp.asarray((lane % (L * NPTS)) // NPTS)
    wl_lane = jnp.asarray(np.array([w for _, w in FM_SHAPES], np.float32))[lidx]
    hl_lane = jnp.asarray(np.array([h for h, _ in FM_SHAPES], np.float32))[lidx]
    pm = projection_mat[0]
    rows = [pm[cidx, i, j] for i in range(3) for j in range(4)]
    xs = wl_lane / image_wh[0, cidx, 0]
    ys = hl_lane / image_wh[0, cidx, 1]
    cf = jnp.stack(rows + [xs, ys, wl_lane, hl_lane], axis=0).astype(f32)
    base1 = (cidx * ROWS_PER_CAM + jnp.asarray(np.array(LEVEL_OFF))[lidx] + 1)
    ci = jnp.stack([base1, wl_lane.astype(jnp.int32),
                    hl_lane.astype(jnp.int32)], axis=0).astype(jnp.int32)

    idx0, idx1, b00, b10, b01, b11, attn6 = _prologue(
        if_p, ae_p, anc_p, wl_p, bl_p, ww_p, bw_p, cf, ci)
    del idx0, idx1, b00, b10, b01, b11
    learn_h = jax.nn.sigmoid(instance_feature @ learnable_fc_w + learnable_fc_b).reshape(bs, na, NLEARN, 3) - 0.5
    fixj = jnp.asarray(FIX_SCALE)
    scale_h = jnp.concatenate([jnp.broadcast_to(fixj[None, None], (bs, na, 7, 3)), learn_h], axis=2)
    kp_h = scale_h * jnp.exp(anchor[:, :, None, 3:6])
    sn_h = anchor[..., 6]
    cs_h = anchor[..., 7]
    zz = jnp.zeros_like(sn_h)
    oo = jnp.ones_like(sn_h)
    Rm = jnp.stack([cs_h, -sn_h, zz, sn_h, cs_h, zz, zz, zz, oo], axis=-1).reshape(bs, na, 3, 3)
    kp_h = jnp.einsum('bnij,bnpj->bnpi', Rm, kp_h) + anchor[:, :, None, :3]
    pts4_h = jnp.concatenate([kp_h, jnp.ones_like(kp_h[..., :1])], axis=-1)
    p2d_h = jnp.einsum('bcij,bnpj->bcnpi', projection_mat, pts4_h)
    p2d_h = p2d_h[..., :2] / jnp.maximum(p2d_h[..., 2:3], 1e-5)
    p2d_h = p2d_h / image_wh[:, :, None, None, :]
    pxh = p2d_h[0, ..., 0]
    pyh = p2d_h[0, ..., 1]
    camb = (jnp.arange(NC, dtype=jnp.int32) * ROWS_PER_CAM)[:, None, None]
    il, blq = [], []
    for l, (H, W) in enumerate(FM_SHAPES):
        x = pxh * W - 0.5
        y = pyh * H - 0.5
        x0 = jnp.floor(x)
        y0 = jnp.floor(y)
        wx1 = x - x0
        wx0 = 1 - wx1
        wy1 = y - y0
        wy0 = 1 - wy1
        vx0 = ((x0 >= 0) & (x0 <= W - 1)).astype(jnp.float32)
        vx1 = ((x0 >= -1) & (x0 <= W - 2)).astype(jnp.float32)
        vy0 = ((y0 >= 0) & (y0 <= H - 1)).astype(jnp.float32)
        vy1 = ((y0 >= -1) & (y0 <= H - 2)).astype(jnp.float32)
        x0s = jnp.clip(x0, -1, W - 1).astype(jnp.int32)
        y0s = jnp.clip(y0, 0, H - 1).astype(jnp.int32)
        y1s = jnp.clip(y0 + 1, 0, H - 1).astype(jnp.int32)
        basej = camb + LEVEL_OFF[l] + 1
        il.append(jnp.stack([basej + y0s * W + x0s, basej + y1s * W + x0s], -1))
        blq.append(jnp.stack([wx0 * wy0 * vx0 * vy0, wx1 * wy0 * vx1 * vy0,
                              wx0 * wy1 * vx0 * vy1, wx1 * wy1 * vx1 * vy1], -1))
    idx_h = jnp.stack(il, 1).transpose(2, 4, 0, 1, 3).reshape(na, 2, S_TOT)
    bil_h = jnp.stack(blq, 1).transpose(2, 4, 0, 1, 3).reshape(na, 4, S_TOT)
    idx6 = jnp.pad(idx_h, ((0, NA_TC - na), (0, 0), (0, 0)))
    bil6 = jnp.pad(bil_h, ((0, NA_TC - na), (0, 0), (0, 0)))
    idxh = idx6[:NA_PAD].reshape(NA_PAD, 2 * NCHUNK, CH_SAMP)
    bilh = bil6[:NA_PAD]
    attnh = attn6[:NA_PAD]

    f_perm = _sc_fuse(table2, idxh, bilh, attnh)

    w_perm = output_proj_w[jnp.asarray(PERM_OUT), :]
    out = _epilogue(f_perm, w_perm, output_proj_b, if_p[:NA_PAD])
    return out[:na].reshape(bs, na, EMBED)
